# baseline (speedup 1.0000x reference)
import functools

import jax
import jax.numpy as jnp
from jax import lax
from jax.experimental import pallas as pl
from jax.experimental.pallas import tpu as pltpu
from jax.experimental.pallas import tpu_sc as plsc

D = 32
VOCAB_N = 1_000_000
BATCH = 4096
SEQ = 200
N = BATCH * SEQ
NW = 32
PER_W = N // NW
CHUNK = 128
NCHUNK = PER_W // CHUNK
K_WAVE = 8
NWAVE = NCHUNK // K_WAVE

BLKC = 8192
NBLK = pl.cdiv(VOCAB_N, BLKC)
QBLK = BLKC // 4
VPAD = NBLK * BLKC


def _proj_body(t1_ref, t2_ref, y1_ref, y2_ref, b_ref, p1_ref, p2_ref):
    dn = (((0,), (0,)), ((), ()))
    x1 = jnp.concatenate(
        [t1_ref[:, u * QBLK:(u + 1) * QBLK] for u in range(4)], axis=0)
    x2 = jnp.concatenate(
        [t2_ref[:, u * QBLK:(u + 1) * QBLK] for u in range(4)], axis=0)
    p1 = lax.dot_general(x1, y1_ref[...], dn,
                         preferred_element_type=jnp.float32)
    p2 = lax.dot_general(x2, y2_ref[...], dn,
                         preferred_element_type=jnp.float32)
    p1_ref[...] = p1
    p2_ref[...] = p2 + b_ref[...]


_proj = pl.pallas_call(
    _proj_body,
    grid=(NBLK,),
    in_specs=[
        pl.BlockSpec((D, BLKC), lambda i: (0, i)),
        pl.BlockSpec((D, BLKC), lambda i: (0, i)),
        pl.BlockSpec((128, 128), lambda i: (0, 0)),
        pl.BlockSpec((128, 128), lambda i: (0, 0)),
        pl.BlockSpec((1, 128), lambda i: (0, 0)),
    ],
    out_specs=[
        pl.BlockSpec((QBLK, 128), lambda i: (i, 0)),
        pl.BlockSpec((QBLK, 128), lambda i: (i, 0)),
    ],
    out_shape=[
        jax.ShapeDtypeStruct((VPAD // 4, 128), jnp.float32),
        jax.ShapeDtypeStruct((VPAD // 4, 128), jnp.float32),
    ],
)

_mesh = plsc.VectorSubcoreMesh(core_axis_name="c", subcore_axis_name="s")

KW = 8
NWAVE2 = SEQ // KW


@functools.partial(
    pl.kernel,
    out_type=jax.ShapeDtypeStruct((SEQ, 4, NW, 8, CHUNK), jnp.float32),
    mesh=_mesh,
    scratch_types=[
        pltpu.VMEM((2, KW, CHUNK), jnp.int32),
        pltpu.VMEM((2, KW, CHUNK), jnp.int32),
        pltpu.VMEM((2, KW, CHUNK, D), jnp.float32),
        pltpu.VMEM((KW, 4, 8, 129), jnp.float32),
        pltpu.SemaphoreType.DMA,
        pltpu.SemaphoreType.DMA,
        pltpu.SemaphoreType.DMA,
        pltpu.SemaphoreType.DMA,
        pltpu.SemaphoreType.DMA,
    ],
    compiler_params=pltpu.CompilerParams(use_tc_tiling_on_sc=False,
                                         needs_layout_passes=False),
)
def _sc_gather_add(p1, p2, ids, eids, out, idx_v, eidx_v, bufs, tbufs,
                   isemA, isemB, gsem1, gsem2, wsem):
    wid = lax.axis_index("s") * 2 + lax.axis_index("c")
    bcol = pl.ds(wid * CHUNK, CHUNK)

    iota16 = lax.iota(jnp.int32, 16)
    qa, sa = iota16 // 8, iota16 % 8
    qb, sb = (iota16 + 16) // 8, (iota16 + 16) % 8

    def fire_idx(wv, sem, aset):
        rows = pl.ds(wv * KW, KW)
        pltpu.async_copy(ids.at[rows, bcol], idx_v.at[aset], sem)
        pltpu.async_copy(eids.at[rows, bcol], eidx_v.at[aset], sem)

    def drain_idx(sem):
        pltpu.make_async_copy(ids.at[pl.ds(0, KW), pl.ds(0, CHUNK)],
                              idx_v.at[0], sem).wait()
        pltpu.make_async_copy(eids.at[pl.ds(0, KW), pl.ds(0, CHUNK)],
                              eidx_v.at[0], sem).wait()

    fire_idx(0, isemA, 0)

    def wave(w, _):
        a = lax.rem(w, 2)
        b2 = 1 - a

        @pl.when(jnp.logical_and(w < NWAVE2 - 1, a == 0))
        def _():
            fire_idx(w + 1, isemB, 1)

        @pl.when(jnp.logical_and(w < NWAVE2 - 1, a == 1))
        def _():
            fire_idx(w + 1, isemA, 0)

        @pl.when(w < NWAVE2)
        def _():
            @pl.when(a == 0)
            def _():
                drain_idx(isemA)

            @pl.when(a == 1)
            def _():
                drain_idx(isemB)

            for u in range(KW):
                pltpu.async_copy(p1.at[idx_v.at[a, u]],
                                 bufs.at[a, u], gsem1)

        @pl.when(w >= 1)
        def _():
            for u in range(KW):
                pltpu.make_async_copy(
                    p2.at[idx_v.at[0, 0]], bufs.at[0, 0], gsem2).wait()

            @pl.when(w >= 2)
            def _():
                for u in range(KW):
                    pltpu.make_async_copy(
                        tbufs.at[0, :, :, pl.ds(0, CHUNK)],
                        out.at[0, :, 0], wsem).wait()

            for u in range(KW):
                bu = bufs.at[b2, u]
                tu = tbufs.at[u]

                def per_m(m, _, bu=bu, tu=tu):
                    va = bu[m, pl.ds(0, 16)]
                    vb = bu[m, pl.ds(16, 16)]
                    m_idx = jnp.full((16,), m, jnp.int32)
                    plsc.store_scatter(tu, [qa, sa, m_idx], va)
                    plsc.store_scatter(tu, [qb, sb, m_idx], vb)
                    return 0

                lax.fori_loop(0, CHUNK, per_m, 0, unroll=16)
            lchunk = (w - 1) * KW
            for u in range(KW):
                pltpu.async_copy(
                    tbufs.at[u, :, :, pl.ds(0, CHUNK)],
                    out.at[lchunk + u, :, wid],
                    wsem,
                )

        @pl.when(w < NWAVE2)
        def _():
            for u in range(KW):
                pltpu.make_async_copy(
                    p1.at[idx_v.at[0, 0]], bufs.at[0, 0], gsem1).wait()
            for u in range(KW):
                pltpu.async_copy(p2.at[eidx_v.at[a, u]],
                                 bufs.at[a, u], gsem2, add=True)
        return 0

    lax.fori_loop(0, NWAVE2 + 1, wave, 0)

    for u in range(KW):
        pltpu.make_async_copy(tbufs.at[0, :, :, pl.ds(0, CHUNK)],
                              out.at[0, :, 0], wsem).wait()


def _pack_ids(raw):
    r = raw.astype(jnp.int32)
    m = ((r >> 13) << 13) + ((r & (QBLK - 1)) << 2) + ((r >> 11) & 3)
    return m.T


def kernel(input_ids, entity_ids, token_table, lkg_table, W, b):
    ids = _pack_ids(input_ids)
    eids = _pack_ids(entity_ids)
    eye4 = jnp.eye(4, dtype=jnp.float32)
    y1 = jnp.kron(eye4, W[:, :D].T)
    y2 = jnp.kron(eye4, W[:, D:].T)
    b128 = jnp.tile(b, 4).reshape(1, 128)
    p1p, p2p = _proj(token_table.T, lkg_table.T, y1, y2, b128)
    p1 = p1p.reshape(VPAD, D)
    p2 = p2p.reshape(VPAD, D)
    out = _sc_gather_add(p1, p2, ids, eids)
    return out.transpose(2, 4, 0, 1, 3).reshape(BATCH, SEQ, D)


# speedup vs baseline: 1.2081x; 1.2081x over previous
import functools

import jax
import jax.numpy as jnp
from jax import lax
from jax.experimental import pallas as pl
from jax.experimental.pallas import tpu as pltpu
from jax.experimental.pallas import tpu_sc as plsc

D = 32
VOCAB_N = 1_000_000
BATCH = 4096
SEQ = 200
N = BATCH * SEQ
NW = 32
PER_W = N // NW
CHUNK = 128
NCHUNK = PER_W // CHUNK
K_WAVE = 8
NWAVE = NCHUNK // K_WAVE

BLKC = 8192
NBLK = pl.cdiv(VOCAB_N, BLKC)
QBLK = BLKC // 4
VPAD = NBLK * BLKC


def _proj_body(t1_ref, t2_ref, y1_ref, y2_ref, b_ref, p1_ref, p2_ref):
    dn = (((0,), (0,)), ((), ()))
    x1 = jnp.concatenate(
        [t1_ref[:, u * QBLK:(u + 1) * QBLK] for u in range(4)], axis=0)
    x2 = jnp.concatenate(
        [t2_ref[:, u * QBLK:(u + 1) * QBLK] for u in range(4)], axis=0)
    p1 = lax.dot_general(x1, y1_ref[...], dn,
                         preferred_element_type=jnp.float32)
    p2 = lax.dot_general(x2, y2_ref[...], dn,
                         preferred_element_type=jnp.float32)
    p1_ref[...] = p1
    p2_ref[...] = p2 + b_ref[...]


_proj = pl.pallas_call(
    _proj_body,
    grid=(NBLK,),
    in_specs=[
        pl.BlockSpec((D, BLKC), lambda i: (0, i)),
        pl.BlockSpec((D, BLKC), lambda i: (0, i)),
        pl.BlockSpec((128, 128), lambda i: (0, 0)),
        pl.BlockSpec((128, 128), lambda i: (0, 0)),
        pl.BlockSpec((1, 128), lambda i: (0, 0)),
    ],
    out_specs=[
        pl.BlockSpec((QBLK, 128), lambda i: (i, 0)),
        pl.BlockSpec((QBLK, 128), lambda i: (i, 0)),
    ],
    out_shape=[
        jax.ShapeDtypeStruct((VPAD // 4, 128), jnp.float32),
        jax.ShapeDtypeStruct((VPAD // 4, 128), jnp.float32),
    ],
)

_mesh = plsc.VectorSubcoreMesh(core_axis_name="c", subcore_axis_name="s")

KW = 4
NWAVE2 = SEQ // KW


@functools.partial(
    pl.kernel,
    out_type=jax.ShapeDtypeStruct((SEQ, 4, NW, 8, CHUNK), jnp.float32),
    mesh=_mesh,
    scratch_types=[
        pltpu.VMEM((SEQ, CHUNK), jnp.int32),
        pltpu.VMEM((SEQ, CHUNK), jnp.int32),
        pltpu.VMEM((3, KW, CHUNK, D), jnp.float32),
        pltpu.VMEM((KW, 4, 8, 129), jnp.float32),
        pltpu.SemaphoreType.DMA,
        pltpu.SemaphoreType.DMA,
        pltpu.SemaphoreType.DMA,
    ],
    compiler_params=pltpu.CompilerParams(use_tc_tiling_on_sc=False,
                                         needs_layout_passes=False),
)
def _sc_gather_add(p1, p2, ids, eids, out, idx_v, eidx_v, bufs, tbufs,
                   gsem1, gsem2, wsem):
    wid = lax.axis_index("s") * 2 + lax.axis_index("c")
    pltpu.sync_copy(ids.at[:, pl.ds(wid * CHUNK, CHUNK)], idx_v)
    pltpu.sync_copy(eids.at[:, pl.ds(wid * CHUNK, CHUNK)], eidx_v)

    iota16 = lax.iota(jnp.int32, 16)
    qa, sa = iota16 // 8, iota16 % 8
    qb, sb = (iota16 + 16) // 8, (iota16 + 16) % 8

    def drain(sem, n):
        for _u in range(n):
            pltpu.make_async_copy(
                p1.at[idx_v.at[0]], bufs.at[0, 0], sem).wait()

    for u in range(KW):
        pltpu.async_copy(p1.at[idx_v.at[u]], bufs.at[0, u], gsem1)

    def wave(w, _):
        rw = lax.rem(w, 3)
        rw1 = lax.rem(w + 1, 3)
        rwm = lax.rem(w + 2, 3)

        @pl.when(w < NWAVE2)
        def _():
            drain(gsem1, KW)

            @pl.when(w + 1 < NWAVE2)
            def _():
                for u in range(KW):
                    pltpu.async_copy(p1.at[idx_v.at[(w + 1) * KW + u]],
                                     bufs.at[rw1, u], gsem1)

            @pl.when(w >= 1)
            def _():
                drain(gsem2, KW)
            for u in range(KW):
                pltpu.async_copy(p2.at[eidx_v.at[w * KW + u]],
                                 bufs.at[rw, u], gsem2, add=True)

        @pl.when(w == NWAVE2)
        def _():
            drain(gsem2, KW)

        @pl.when(w >= 1)
        def _():
            @pl.when(w >= 2)
            def _():
                for u in range(KW):
                    pltpu.make_async_copy(
                        tbufs.at[0, :, :, pl.ds(0, CHUNK)],
                        out.at[0, :, 0], wsem).wait()

            for u in range(KW):
                bu = bufs.at[rwm, u]
                tu = tbufs.at[u]

                def per_m(m, _, bu=bu, tu=tu):
                    va = bu[m, pl.ds(0, 16)]
                    vb = bu[m, pl.ds(16, 16)]
                    m_idx = jnp.full((16,), m, jnp.int32)
                    plsc.store_scatter(tu, [qa, sa, m_idx], va)
                    plsc.store_scatter(tu, [qb, sb, m_idx], vb)
                    return 0

                lax.fori_loop(0, CHUNK, per_m, 0, unroll=16)
            lchunk = (w - 1) * KW
            for u in range(KW):
                pltpu.async_copy(
                    tbufs.at[u, :, :, pl.ds(0, CHUNK)],
                    out.at[lchunk + u, :, wid],
                    wsem,
                )
        return 0

    lax.fori_loop(0, NWAVE2 + 1, wave, 0)

    for u in range(KW):
        pltpu.make_async_copy(tbufs.at[0, :, :, pl.ds(0, CHUNK)],
                              out.at[0, :, 0], wsem).wait()


def _pack_ids(raw):
    r = raw.astype(jnp.int32)
    m = ((r >> 13) << 13) + ((r & (QBLK - 1)) << 2) + ((r >> 11) & 3)
    return m.T


def kernel(input_ids, entity_ids, token_table, lkg_table, W, b):
    ids = _pack_ids(input_ids)
    eids = _pack_ids(entity_ids)
    eye4 = jnp.eye(4, dtype=jnp.float32)
    y1 = jnp.kron(eye4, W[:, :D].T)
    y2 = jnp.kron(eye4, W[:, D:].T)
    b128 = jnp.tile(b, 4).reshape(1, 128)
    p1p, p2p = _proj(token_table.T, lkg_table.T, y1, y2, b128)
    p1 = p1p.reshape(VPAD, D)
    p2 = p2p.reshape(VPAD, D)
    out = _sc_gather_add(p1, p2, ids, eids)
    return out.transpose(2, 4, 0, 1, 3).reshape(BATCH, SEQ, D)


# speedup vs baseline: 1.3013x; 1.0771x over previous
import functools

import jax
import jax.numpy as jnp
from jax import lax
from jax.experimental import pallas as pl
from jax.experimental.pallas import tpu as pltpu
from jax.experimental.pallas import tpu_sc as plsc

D = 32
VOCAB_N = 1_000_000
BATCH = 4096
SEQ = 200
N = BATCH * SEQ
NW = 32
PER_W = N // NW
CHUNK = 128
NCHUNK = PER_W // CHUNK
K_WAVE = 8
NWAVE = NCHUNK // K_WAVE

BLKC = 16384
NBLK = pl.cdiv(VOCAB_N, BLKC)
QBLK = BLKC // 4
VPAD = NBLK * BLKC


def _proj_body(t1_ref, t2_ref, y1_ref, y2_ref, b_ref, p1_ref, p2_ref):
    dn = (((0,), (0,)), ((), ()))
    x1 = jnp.concatenate(
        [t1_ref[:, u * QBLK:(u + 1) * QBLK] for u in range(4)], axis=0)
    x2 = jnp.concatenate(
        [t2_ref[:, u * QBLK:(u + 1) * QBLK] for u in range(4)], axis=0)
    p1 = lax.dot_general(x1, y1_ref[...], dn,
                         preferred_element_type=jnp.float32)
    p2 = lax.dot_general(x2, y2_ref[...], dn,
                         preferred_element_type=jnp.float32)
    p1_ref[...] = p1
    p2_ref[...] = p2 + b_ref[...]


_proj = pl.pallas_call(
    _proj_body,
    grid=(NBLK,),
    in_specs=[
        pl.BlockSpec((D, BLKC), lambda i: (0, i)),
        pl.BlockSpec((D, BLKC), lambda i: (0, i)),
        pl.BlockSpec((128, 128), lambda i: (0, 0)),
        pl.BlockSpec((128, 128), lambda i: (0, 0)),
        pl.BlockSpec((1, 128), lambda i: (0, 0)),
    ],
    out_specs=[
        pl.BlockSpec((QBLK, 128), lambda i: (i, 0)),
        pl.BlockSpec((QBLK, 128), lambda i: (i, 0)),
    ],
    out_shape=[
        jax.ShapeDtypeStruct((VPAD // 4, 128), jnp.float32),
        jax.ShapeDtypeStruct((VPAD // 4, 128), jnp.float32),
    ],
)

_mesh = plsc.VectorSubcoreMesh(core_axis_name="c", subcore_axis_name="s")

KW = 4
NWAVE2 = SEQ // KW


@functools.partial(
    pl.kernel,
    out_type=jax.ShapeDtypeStruct((SEQ, 4, NW, 8, CHUNK), jnp.float32),
    mesh=_mesh,
    scratch_types=[
        pltpu.VMEM((SEQ, CHUNK), jnp.int32),
        pltpu.VMEM((SEQ, CHUNK), jnp.int32),
        pltpu.VMEM((3, KW, CHUNK, D), jnp.float32),
        pltpu.VMEM((KW, 4, 8, 129), jnp.float32),
        pltpu.SemaphoreType.DMA,
        pltpu.SemaphoreType.DMA,
        pltpu.SemaphoreType.DMA,
    ],
    compiler_params=pltpu.CompilerParams(use_tc_tiling_on_sc=False,
                                         needs_layout_passes=False),
)
def _sc_gather_add(p1, p2, ids, eids, out, idx_v, eidx_v, bufs, tbufs,
                   gsem1, gsem2, wsem):
    wid = lax.axis_index("s") * 2 + lax.axis_index("c")
    pltpu.sync_copy(ids.at[:, pl.ds(wid * CHUNK, CHUNK)], idx_v)
    pltpu.sync_copy(eids.at[:, pl.ds(wid * CHUNK, CHUNK)], eidx_v)

    iota16 = lax.iota(jnp.int32, 16)
    qa, sa = iota16 // 8, iota16 % 8
    qb, sb = (iota16 + 16) // 8, (iota16 + 16) % 8

    def drain(sem, n):
        for _u in range(n):
            pltpu.make_async_copy(
                p1.at[idx_v.at[0]], bufs.at[0, 0], sem).wait()

    for u in range(KW):
        pltpu.async_copy(p1.at[idx_v.at[u]], bufs.at[0, u], gsem1)

    def wave(w, _):
        rw = lax.rem(w, 3)
        rw1 = lax.rem(w + 1, 3)
        rwm = lax.rem(w + 2, 3)

        @pl.when(w < NWAVE2)
        def _():
            drain(gsem1, KW)

            @pl.when(w + 1 < NWAVE2)
            def _():
                for u in range(KW):
                    pltpu.async_copy(p1.at[idx_v.at[(w + 1) * KW + u]],
                                     bufs.at[rw1, u], gsem1)

            @pl.when(w >= 1)
            def _():
                drain(gsem2, KW)
            for u in range(KW):
                pltpu.async_copy(p2.at[eidx_v.at[w * KW + u]],
                                 bufs.at[rw, u], gsem2, add=True)

        @pl.when(w == NWAVE2)
        def _():
            drain(gsem2, KW)

        @pl.when(w >= 1)
        def _():
            @pl.when(w >= 2)
            def _():
                for u in range(KW):
                    pltpu.make_async_copy(
                        tbufs.at[0, :, :, pl.ds(0, CHUNK)],
                        out.at[0, :, 0], wsem).wait()

            for u in range(KW):
                bu = bufs.at[rwm, u]
                tu = tbufs.at[u]

                def per_m(m, _, bu=bu, tu=tu):
                    va = bu[m, pl.ds(0, 16)]
                    vb = bu[m, pl.ds(16, 16)]
                    m_idx = jnp.full((16,), m, jnp.int32)
                    plsc.store_scatter(tu, [qa, sa, m_idx], va)
                    plsc.store_scatter(tu, [qb, sb, m_idx], vb)
                    return 0

                lax.fori_loop(0, CHUNK, per_m, 0, unroll=16)
            lchunk = (w - 1) * KW
            for u in range(KW):
                pltpu.async_copy(
                    tbufs.at[u, :, :, pl.ds(0, CHUNK)],
                    out.at[lchunk + u, :, wid],
                    wsem,
                )
        return 0

    lax.fori_loop(0, NWAVE2 + 1, wave, 0)

    for u in range(KW):
        pltpu.make_async_copy(tbufs.at[0, :, :, pl.ds(0, CHUNK)],
                              out.at[0, :, 0], wsem).wait()


def _pack_ids(raw):
    sh = BLKC.bit_length() - 1
    r = raw.astype(jnp.int32)
    m = ((r >> sh) << sh) + ((r & (QBLK - 1)) << 2) + ((r >> (sh - 2)) & 3)
    return m.T


def kernel(input_ids, entity_ids, token_table, lkg_table, W, b):
    ids = _pack_ids(input_ids)
    eids = _pack_ids(entity_ids)
    eye4 = jnp.eye(4, dtype=jnp.float32)
    y1 = jnp.kron(eye4, W[:, :D].T)
    y2 = jnp.kron(eye4, W[:, D:].T)
    b128 = jnp.tile(b, 4).reshape(1, 128)
    p1p, p2p = _proj(token_table.T, lkg_table.T, y1, y2, b128)
    p1 = p1p.reshape(VPAD, D)
    p2 = p2p.reshape(VPAD, D)
    out = _sc_gather_add(p1, p2, ids, eids)
    return out.transpose(2, 4, 0, 1, 3).reshape(BATCH, SEQ, D)


# speedup vs baseline: 1.3160x; 1.0113x over previous
import functools

import jax
import jax.numpy as jnp
from jax import lax
from jax.experimental import pallas as pl
from jax.experimental.pallas import tpu as pltpu
from jax.experimental.pallas import tpu_sc as plsc

D = 32
VOCAB_N = 1_000_000
BATCH = 4096
SEQ = 200
N = BATCH * SEQ
NW = 32
PER_W = N // NW
CHUNK = 128
NCHUNK = PER_W // CHUNK
K_WAVE = 8
NWAVE = NCHUNK // K_WAVE

BLKC = 32768
NBLK = pl.cdiv(VOCAB_N, BLKC)
QBLK = BLKC // 4
VPAD = NBLK * BLKC


def _proj_body(t1_ref, t2_ref, y1_ref, y2_ref, b_ref, p1_ref, p2_ref):
    dn = (((0,), (0,)), ((), ()))
    x1 = jnp.concatenate(
        [t1_ref[:, u * QBLK:(u + 1) * QBLK] for u in range(4)], axis=0)
    x2 = jnp.concatenate(
        [t2_ref[:, u * QBLK:(u + 1) * QBLK] for u in range(4)], axis=0)
    p1 = lax.dot_general(x1, y1_ref[...], dn,
                         preferred_element_type=jnp.float32)
    p2 = lax.dot_general(x2, y2_ref[...], dn,
                         preferred_element_type=jnp.float32)
    p1_ref[...] = p1
    p2_ref[...] = p2 + b_ref[...]


_proj = pl.pallas_call(
    _proj_body,
    grid=(NBLK,),
    in_specs=[
        pl.BlockSpec((D, BLKC), lambda i: (0, i)),
        pl.BlockSpec((D, BLKC), lambda i: (0, i)),
        pl.BlockSpec((128, 128), lambda i: (0, 0)),
        pl.BlockSpec((128, 128), lambda i: (0, 0)),
        pl.BlockSpec((1, 128), lambda i: (0, 0)),
    ],
    out_specs=[
        pl.BlockSpec((QBLK, 128), lambda i: (i, 0)),
        pl.BlockSpec((QBLK, 128), lambda i: (i, 0)),
    ],
    out_shape=[
        jax.ShapeDtypeStruct((VPAD // 4, 128), jnp.float32),
        jax.ShapeDtypeStruct((VPAD // 4, 128), jnp.float32),
    ],
)

_mesh = plsc.VectorSubcoreMesh(core_axis_name="c", subcore_axis_name="s")

KW = 4
NWAVE2 = SEQ // KW


@functools.partial(
    pl.kernel,
    out_type=jax.ShapeDtypeStruct((SEQ, 4, NW, 8, CHUNK), jnp.float32),
    mesh=_mesh,
    scratch_types=[
        pltpu.VMEM((SEQ, CHUNK), jnp.int32),
        pltpu.VMEM((SEQ, CHUNK), jnp.int32),
        pltpu.VMEM((3, KW, CHUNK, D), jnp.float32),
        pltpu.VMEM((KW, 4, 8, 129), jnp.float32),
        pltpu.SemaphoreType.DMA,
        pltpu.SemaphoreType.DMA,
        pltpu.SemaphoreType.DMA,
    ],
    compiler_params=pltpu.CompilerParams(use_tc_tiling_on_sc=False,
                                         needs_layout_passes=False),
)
def _sc_gather_add(p1, p2, ids, eids, out, idx_v, eidx_v, bufs, tbufs,
                   gsem1, gsem2, wsem):
    wid = lax.axis_index("s") * 2 + lax.axis_index("c")
    pltpu.sync_copy(ids.at[:, pl.ds(wid * CHUNK, CHUNK)], idx_v)
    pltpu.sync_copy(eids.at[:, pl.ds(wid * CHUNK, CHUNK)], eidx_v)

    iota16 = lax.iota(jnp.int32, 16)
    qa, sa = iota16 // 8, iota16 % 8
    qb, sb = (iota16 + 16) // 8, (iota16 + 16) % 8

    def drain(sem, n):
        for _u in range(n):
            pltpu.make_async_copy(
                p1.at[idx_v.at[0]], bufs.at[0, 0], sem).wait()

    for u in range(KW):
        pltpu.async_copy(p1.at[idx_v.at[u]], bufs.at[0, u], gsem1)

    def wave(w, _):
        rw = lax.rem(w, 3)
        rw1 = lax.rem(w + 1, 3)
        rwm = lax.rem(w + 2, 3)

        @pl.when(w < NWAVE2)
        def _():
            drain(gsem1, KW)

            @pl.when(w + 1 < NWAVE2)
            def _():
                for u in range(KW):
                    pltpu.async_copy(p1.at[idx_v.at[(w + 1) * KW + u]],
                                     bufs.at[rw1, u], gsem1)

            @pl.when(w >= 1)
            def _():
                drain(gsem2, KW)
            for u in range(KW):
                pltpu.async_copy(p2.at[eidx_v.at[w * KW + u]],
                                 bufs.at[rw, u], gsem2, add=True)

        @pl.when(w == NWAVE2)
        def _():
            drain(gsem2, KW)

        @pl.when(w >= 1)
        def _():
            @pl.when(w >= 2)
            def _():
                for u in range(KW):
                    pltpu.make_async_copy(
                        tbufs.at[0, :, :, pl.ds(0, CHUNK)],
                        out.at[0, :, 0], wsem).wait()

            for u in range(KW):
                bu = bufs.at[rwm, u]
                tu = tbufs.at[u]

                def per_m(m, _, bu=bu, tu=tu):
                    va = bu[m, pl.ds(0, 16)]
                    vb = bu[m, pl.ds(16, 16)]
                    m_idx = jnp.full((16,), m, jnp.int32)
                    plsc.store_scatter(tu, [qa, sa, m_idx], va)
                    plsc.store_scatter(tu, [qb, sb, m_idx], vb)
                    return 0

                lax.fori_loop(0, CHUNK, per_m, 0, unroll=16)
            lchunk = (w - 1) * KW
            for u in range(KW):
                pltpu.async_copy(
                    tbufs.at[u, :, :, pl.ds(0, CHUNK)],
                    out.at[lchunk + u, :, wid],
                    wsem,
                )
        return 0

    lax.fori_loop(0, NWAVE2 + 1, wave, 0)

    for u in range(KW):
        pltpu.make_async_copy(tbufs.at[0, :, :, pl.ds(0, CHUNK)],
                              out.at[0, :, 0], wsem).wait()


def _pack_ids(raw):
    sh = BLKC.bit_length() - 1
    r = raw.astype(jnp.int32)
    m = ((r >> sh) << sh) + ((r & (QBLK - 1)) << 2) + ((r >> (sh - 2)) & 3)
    return m.T


def kernel(input_ids, entity_ids, token_table, lkg_table, W, b):
    ids = _pack_ids(input_ids)
    eids = _pack_ids(entity_ids)
    eye4 = jnp.eye(4, dtype=jnp.float32)
    y1 = jnp.kron(eye4, W[:, :D].T)
    y2 = jnp.kron(eye4, W[:, D:].T)
    b128 = jnp.tile(b, 4).reshape(1, 128)
    p1p, p2p = _proj(token_table.T, lkg_table.T, y1, y2, b128)
    p1 = p1p.reshape(VPAD, D)
    p2 = p2p.reshape(VPAD, D)
    out = _sc_gather_add(p1, p2, ids, eids)
    return out.transpose(2, 4, 0, 1, 3).reshape(BATCH, SEQ, D)
